# baseline (speedup 1.0000x reference)
import functools

import jax
import jax.numpy as jnp
from jax import lax
from jax.experimental import pallas as pl
from jax.experimental.pallas import tpu as pltpu
from jax.experimental.pallas import tpu_sc as plsc

B = 16384
F = 26
V = 100000
D = 16
ND = 13

NC = 2
NS = 16
NW = NC * NS

B_PER_W = B // NW
CHUNK = 128
N_CHUNK = B_PER_W // CHUNK


def _gather_body(tbl_hbm, idx_hbm, out_hbm, idx_v, rows_v, sem):
    wid = lax.axis_index("s") * NC + lax.axis_index("c")
    b0 = wid * B_PER_W
    pltpu.sync_copy(idx_hbm.at[:, pl.ds(b0, B_PER_W)], idx_v)

    def field(f, carry):
        copies = []
        for j in range(N_CHUNK):
            copies.append(
                pltpu.async_copy(
                    tbl_hbm.at[f].at[idx_v.at[f, pl.ds(j * CHUNK, CHUNK)]],
                    rows_v.at[pl.ds(j * CHUNK, CHUNK)],
                    sem,
                )
            )
        for c in copies:
            c.wait()
        pltpu.sync_copy(rows_v, out_hbm.at[pl.ds(b0, B_PER_W), pl.ds(f * D, D)])
        return carry

    lax.fori_loop(0, F, field, 0)


def _sc_gather(tables, inputs_sparse):
    mesh = plsc.VectorSubcoreMesh(core_axis_name="c", subcore_axis_name="s")
    return pl.kernel(
        _gather_body,
        out_type=jax.ShapeDtypeStruct((B, F * D), jnp.float32),
        mesh=mesh,
        scratch_types=[
            pltpu.VMEM((F, B_PER_W), jnp.int32),
            pltpu.VMEM((B_PER_W, D), jnp.float32),
            pltpu.SemaphoreType.DMA,
        ],
        compiler_params=pltpu.CompilerParams(use_tc_tiling_on_sc=False),
    )(tables, inputs_sparse)


CP_LANES = 256
FULL_CP = V // CP_LANES
TAIL_LANES = V - FULL_CP * CP_LANES
N_FULL = F * FULL_CP
ITERS = -(-N_FULL // NW)
IN_W = CP_LANES + 1
RPF = V // 8
RPF_PAD = 12504
VPAD = RPF_PAD * 8


def _tr_body(tbl_hbm, tail_hbm, out_hbm, in_v, out_v, sem):
    wid = lax.axis_index("s") * NC + lax.axis_index("c")
    lanes = jax.lax.iota(jnp.int32, 16)

    def transpose_rows(n_rows):
        def qloop(r, carry):
            for k in range(8):
                g = plsc.load_gather(in_v, [lanes, jnp.full((16,), 0, jnp.int32) + 8 * r + k])
                out_v[r, pl.ds(16 * k, 16)] = g
            return carry
        lax.fori_loop(0, n_rows, qloop, 0)

    def item(j, carry):
        t = j * NW + wid
        valid = t < N_FULL
        f = t // FULL_CP
        cp = t % FULL_CP

        @pl.when(valid)
        def _():
            pltpu.sync_copy(
                tbl_hbm.at[f, :, pl.ds(cp * CP_LANES, CP_LANES)],
                in_v.at[:, pl.ds(0, CP_LANES)],
            )
            transpose_rows(32)
            pltpu.sync_copy(
                out_v,
                out_hbm.at[pl.ds(f * RPF_PAD + cp * 32, 32)],
            )
        return carry

    lax.fori_loop(0, ITERS, item, 0)

    @pl.when(wid < F)
    def _():
        f = wid
        pltpu.sync_copy(
            tail_hbm.at[f],
            out_hbm.at[pl.ds(f * RPF_PAD + FULL_CP * 32, 24)],
        )


def _transpose_tables(tablesT, tail):
    mesh = plsc.VectorSubcoreMesh(core_axis_name="c", subcore_axis_name="s")
    return pl.kernel(
        _tr_body,
        out_type=jax.ShapeDtypeStruct((F * RPF_PAD, 8 * D), jnp.float32),
        mesh=mesh,
        scratch_types=[
            pltpu.VMEM((D, IN_W), jnp.float32),
            pltpu.VMEM((32, 8 * D), jnp.float32),
            pltpu.SemaphoreType.DMA,
        ],
        compiler_params=pltpu.CompilerParams(
            use_tc_tiling_on_sc=True, needs_layout_passes=False),
    )(tablesT, tail)


def _mlp_body(emb_ref, dense_ref, wbot_ref, bbot_ref, w1a_ref, w1b_ref,
              b1_ref, w2_ref, b2_ref, w3_ref, b3_ref, w4_ref, b4_ref,
              out_ref):
    f32 = jnp.float32
    demb = jnp.dot(dense_ref[...], wbot_ref[...], preferred_element_type=f32)
    demb = demb + bbot_ref[...]
    h = jnp.dot(emb_ref[...], w1a_ref[...], preferred_element_type=f32)
    h = h + jnp.dot(demb, w1b_ref[...], preferred_element_type=f32)
    h = jnp.maximum(h + b1_ref[...], 0.0)
    h = jnp.maximum(jnp.dot(h, w2_ref[...], preferred_element_type=f32) + b2_ref[...], 0.0)
    h = jnp.maximum(jnp.dot(h, w3_ref[...], preferred_element_type=f32) + b3_ref[...], 0.0)
    o = jnp.dot(h, w4_ref[...], preferred_element_type=f32) + b4_ref[...]
    out_ref[...] = jax.nn.sigmoid(o)


_BB = 2048


def _mlp(emb, dense, wbot, bbot, w1a, w1b, b1, w2, b2, w3, b3, w4, b4):
    full = lambda shape: pl.BlockSpec(shape, lambda i: (0, 0))
    return pl.pallas_call(
        _mlp_body,
        grid=(B // _BB,),
        in_specs=[
            pl.BlockSpec((_BB, F * D), lambda i: (i, 0)),
            pl.BlockSpec((_BB, ND), lambda i: (i, 0)),
            full((ND, D)),
            full((1, D)),
            full((F * D, 256)),
            full((D, 256)),
            full((1, 256)),
            full((256, 128)),
            full((1, 128)),
            full((128, 64)),
            full((1, 64)),
            full((64, 1)),
            full((1, 1)),
        ],
        out_specs=pl.BlockSpec((_BB, 1), lambda i: (i, 0)),
        out_shape=jax.ShapeDtypeStruct((B, 1), jnp.float32),
    )(emb, dense, wbot, bbot, w1a, w1b, b1, w2, b2, w3, b3, w4, b4)


def kernel(inputs_sparse, inputs_dense, tables, W_bot, b_bot,
           W1, b1, W2, b2, W3, b3, W4, b4):
    tablesT = jnp.swapaxes(tables, 1, 2)
    tail = jnp.swapaxes(tablesT[:, :, FULL_CP * CP_LANES:], 1, 2)
    tail = jnp.pad(tail.reshape(F, TAIL_LANES // 8, 8 * D),
                   ((0, 0), (0, 4), (0, 0)))
    t3 = _transpose_tables(tablesT, tail).reshape(F, VPAD, D)
    emb = _sc_gather(t3, inputs_sparse.astype(jnp.int32))

    out = _mlp(
        emb, inputs_dense, W_bot, b_bot.reshape(1, D),
        W1[: F * D], W1[F * D:], b1.reshape(1, 256),
        W2, b2.reshape(1, 128), W3, b3.reshape(1, 64),
        W4, b4.reshape(1, 1),
    )
    return out.reshape(-1)


# speedup vs baseline: 1.9792x; 1.9792x over previous
import functools

import jax
import jax.numpy as jnp
from jax import lax
from jax.experimental import pallas as pl
from jax.experimental.pallas import tpu as pltpu
from jax.experimental.pallas import tpu_sc as plsc

B = 16384
F = 26
V = 100000
D = 16
ND = 13

NC = 2
NS = 16
NW = NC * NS

B_PER_W = B // NW
CHUNK = 128
N_CHUNK = B_PER_W // CHUNK


def _gather_body(tbl_hbm, idx_hbm, out_hbm, idx_v, rows_v, sem):
    wid = lax.axis_index("s") * NC + lax.axis_index("c")
    b0 = wid * B_PER_W
    pltpu.sync_copy(idx_hbm.at[:, pl.ds(b0, B_PER_W)], idx_v)

    def field(f, carry):
        copies = []
        for j in range(N_CHUNK):
            copies.append(
                pltpu.async_copy(
                    tbl_hbm.at[f].at[idx_v.at[f, pl.ds(j * CHUNK, CHUNK)]],
                    rows_v.at[pl.ds(j * CHUNK, CHUNK)],
                    sem,
                )
            )
        for c in copies:
            c.wait()
        pltpu.sync_copy(rows_v, out_hbm.at[pl.ds(b0, B_PER_W), pl.ds(f * D, D)])
        return carry

    lax.fori_loop(0, F, field, 0)


def _sc_gather(tables, inputs_sparse):
    mesh = plsc.VectorSubcoreMesh(core_axis_name="c", subcore_axis_name="s")
    return pl.kernel(
        _gather_body,
        out_type=jax.ShapeDtypeStruct((B, F * D), jnp.float32),
        mesh=mesh,
        scratch_types=[
            pltpu.VMEM((F, B_PER_W), jnp.int32),
            pltpu.VMEM((B_PER_W, D), jnp.float32),
            pltpu.SemaphoreType.DMA,
        ],
        compiler_params=pltpu.CompilerParams(use_tc_tiling_on_sc=False),
    )(tables, inputs_sparse)


CP_LANES = 256
FULL_CP = V // CP_LANES
TAIL_LANES = V - FULL_CP * CP_LANES
N_FULL = F * FULL_CP
ITERS = -(-N_FULL // NW)
IN_W = CP_LANES + 1
RPF = V // 8
RPF_PAD = 12504
VPAD = RPF_PAD * 8


def _tr_body(tbl_hbm, tail_hbm, out_hbm,
             in_a, in_b, out_a, out_b, sia, sib, soa, sob):
    wid = lax.axis_index("s") * NC + lax.axis_index("c")
    lanes = jax.lax.iota(jnp.int32, 16)
    zeros = jnp.full((16,), 0, jnp.int32)

    def in_src(t):
        return tbl_hbm.at[t // FULL_CP, :,
                          pl.ds((t % FULL_CP) * CP_LANES, CP_LANES)]

    def out_dst(t):
        return out_hbm.at[pl.ds((t // FULL_CP) * RPF_PAD
                                + (t % FULL_CP) * 32, 32)]

    def compute(in_v, out_v):
        def qloop(r, carry):
            gs = [plsc.load_gather(in_v, [lanes, zeros + (8 * r + k)])
                  for k in range(8)]
            for k in range(8):
                out_v[r, pl.ds(16 * k, 16)] = gs[k]
            return carry
        lax.fori_loop(0, 32, qloop, 0)

    pltpu.async_copy(in_src(wid), in_a.at[:, pl.ds(0, CP_LANES)], sia)

    def dbl(j, carry):
        t0 = (2 * j) * NW + wid
        t1 = t0 + NW
        pltpu.async_copy(in_src(t1), in_b.at[:, pl.ds(0, CP_LANES)], sib)
        pltpu.make_async_copy(in_src(t0),
                              in_a.at[:, pl.ds(0, CP_LANES)], sia).wait()
        @pl.when(j > 0)
        def _():
            pltpu.make_async_copy(out_a, out_dst(t0), soa).wait()
        compute(in_a, out_a)
        pltpu.async_copy(out_a, out_dst(t0), soa)
        @pl.when(j < 157)
        def _():
            pltpu.async_copy(in_src(t0 + 2 * NW),
                             in_a.at[:, pl.ds(0, CP_LANES)], sia)
        pltpu.make_async_copy(in_src(t1),
                              in_b.at[:, pl.ds(0, CP_LANES)], sib).wait()
        @pl.when(j > 0)
        def _():
            pltpu.make_async_copy(out_b, out_dst(t1), sob).wait()
        compute(in_b, out_b)
        pltpu.async_copy(out_b, out_dst(t1), sob)
        return carry

    lax.fori_loop(0, 158, dbl, 0)
    pltpu.make_async_copy(out_a, out_dst(0), soa).wait()
    pltpu.make_async_copy(out_b, out_dst(0), sob).wait()

    t_last = 316 * NW + wid
    @pl.when(t_last < N_FULL)
    def _():
        pltpu.sync_copy(in_src(t_last), in_a.at[:, pl.ds(0, CP_LANES)])
        compute(in_a, out_a)
        pltpu.sync_copy(out_a, out_dst(t_last))

    @pl.when(wid < F)
    def _():
        f = wid
        pltpu.sync_copy(
            tail_hbm.at[f],
            out_hbm.at[pl.ds(f * RPF_PAD + FULL_CP * 32, 24)],
        )


def _transpose_tables(tablesT, tail):
    mesh = plsc.VectorSubcoreMesh(core_axis_name="c", subcore_axis_name="s")
    return pl.kernel(
        _tr_body,
        out_type=jax.ShapeDtypeStruct((F * RPF_PAD, 8 * D), jnp.float32),
        mesh=mesh,
        scratch_types=[
            pltpu.VMEM((D, IN_W), jnp.float32),
            pltpu.VMEM((D, IN_W), jnp.float32),
            pltpu.VMEM((32, 8 * D), jnp.float32),
            pltpu.VMEM((32, 8 * D), jnp.float32),
            pltpu.SemaphoreType.DMA,
            pltpu.SemaphoreType.DMA,
            pltpu.SemaphoreType.DMA,
            pltpu.SemaphoreType.DMA,
        ],
        compiler_params=pltpu.CompilerParams(
            use_tc_tiling_on_sc=True, needs_layout_passes=False),
    )(tablesT, tail)


def _mlp_body(emb_ref, dense_ref, wbot_ref, bbot_ref, w1a_ref, w1b_ref,
              b1_ref, w2_ref, b2_ref, w3_ref, b3_ref, w4_ref, b4_ref,
              out_ref):
    f32 = jnp.float32
    demb = jnp.dot(dense_ref[...], wbot_ref[...], preferred_element_type=f32)
    demb = demb + bbot_ref[...]
    h = jnp.dot(emb_ref[...], w1a_ref[...], preferred_element_type=f32)
    h = h + jnp.dot(demb, w1b_ref[...], preferred_element_type=f32)
    h = jnp.maximum(h + b1_ref[...], 0.0)
    h = jnp.maximum(jnp.dot(h, w2_ref[...], preferred_element_type=f32) + b2_ref[...], 0.0)
    h = jnp.maximum(jnp.dot(h, w3_ref[...], preferred_element_type=f32) + b3_ref[...], 0.0)
    o = jnp.dot(h, w4_ref[...], preferred_element_type=f32) + b4_ref[...]
    out_ref[...] = jax.nn.sigmoid(o)


_BB = 2048


def _mlp(emb, dense, wbot, bbot, w1a, w1b, b1, w2, b2, w3, b3, w4, b4):
    full = lambda shape: pl.BlockSpec(shape, lambda i: (0, 0))
    return pl.pallas_call(
        _mlp_body,
        grid=(B // _BB,),
        in_specs=[
            pl.BlockSpec((_BB, F * D), lambda i: (i, 0)),
            pl.BlockSpec((_BB, ND), lambda i: (i, 0)),
            full((ND, D)),
            full((1, D)),
            full((F * D, 256)),
            full((D, 256)),
            full((1, 256)),
            full((256, 128)),
            full((1, 128)),
            full((128, 64)),
            full((1, 64)),
            full((64, 1)),
            full((1, 1)),
        ],
        out_specs=pl.BlockSpec((_BB, 1), lambda i: (i, 0)),
        out_shape=jax.ShapeDtypeStruct((B, 1), jnp.float32),
    )(emb, dense, wbot, bbot, w1a, w1b, b1, w2, b2, w3, b3, w4, b4)


def kernel(inputs_sparse, inputs_dense, tables, W_bot, b_bot,
           W1, b1, W2, b2, W3, b3, W4, b4):
    tablesT = jnp.swapaxes(tables, 1, 2)
    tail = jnp.swapaxes(tablesT[:, :, FULL_CP * CP_LANES:], 1, 2)
    tail = jnp.pad(tail.reshape(F, TAIL_LANES // 8, 8 * D),
                   ((0, 0), (0, 4), (0, 0)))
    t3 = _transpose_tables(tablesT, tail).reshape(F, VPAD, D)
    emb = _sc_gather(t3, inputs_sparse.astype(jnp.int32))

    out = _mlp(
        emb, inputs_dense, W_bot, b_bot.reshape(1, D),
        W1[: F * D], W1[F * D:], b1.reshape(1, 256),
        W2, b2.reshape(1, 128), W3, b3.reshape(1, 64),
        W4, b4.reshape(1, 1),
    )
    return out.reshape(-1)


# speedup vs baseline: 2.0081x; 1.0146x over previous
import functools

import jax
import jax.numpy as jnp
from jax import lax
from jax.experimental import pallas as pl
from jax.experimental.pallas import tpu as pltpu
from jax.experimental.pallas import tpu_sc as plsc

B = 16384
F = 26
V = 100000
D = 16
ND = 13

NC = 2
NS = 16
NW = NC * NS

B_PER_W = B // NW
CHUNK = 128
N_CHUNK = B_PER_W // CHUNK


def _gather_body(tbl_hbm, idx_hbm, out_hbm, idx_v, rows_v, sem):
    wid = lax.axis_index("s") * NC + lax.axis_index("c")
    b0 = wid * B_PER_W
    pltpu.sync_copy(idx_hbm.at[:, pl.ds(b0, B_PER_W)], idx_v)

    def field(f, carry):
        copies = []
        for j in range(N_CHUNK):
            copies.append(
                pltpu.async_copy(
                    tbl_hbm.at[f].at[idx_v.at[f, pl.ds(j * CHUNK, CHUNK)]],
                    rows_v.at[pl.ds(j * CHUNK, CHUNK)],
                    sem,
                )
            )
        for c in copies:
            c.wait()
        pltpu.sync_copy(rows_v, out_hbm.at[pl.ds(b0, B_PER_W), pl.ds(f * D, D)])
        return carry

    lax.fori_loop(0, F, field, 0)


def _sc_gather(tables, inputs_sparse):
    mesh = plsc.VectorSubcoreMesh(core_axis_name="c", subcore_axis_name="s")
    return pl.kernel(
        _gather_body,
        out_type=jax.ShapeDtypeStruct((B, F * D), jnp.float32),
        mesh=mesh,
        scratch_types=[
            pltpu.VMEM((F, B_PER_W), jnp.int32),
            pltpu.VMEM((B_PER_W, D), jnp.float32),
            pltpu.SemaphoreType.DMA,
        ],
        compiler_params=pltpu.CompilerParams(use_tc_tiling_on_sc=False),
    )(tables, inputs_sparse)


CP_LANES = 512
FULL_CP = V // CP_LANES
RPI = CP_LANES // 8
TAIL_LANES = V - FULL_CP * CP_LANES
N_FULL = F * FULL_CP
MAIN_ITEMS = (N_FULL // NW) // 2 * 2
DBL = MAIN_ITEMS // 2
IN_W = CP_LANES + 1
RPF = V // 8
RPF_PAD = 12504
VPAD = RPF_PAD * 8


def _tr_body(tbl_hbm, tail_hbm, out_hbm,
             in_a, in_b, out_a, out_b, sia, sib, soa, sob):
    wid = lax.axis_index("s") * NC + lax.axis_index("c")
    lanes = jax.lax.iota(jnp.int32, 16)
    zeros = jnp.full((16,), 0, jnp.int32)

    def in_src(t):
        return tbl_hbm.at[t // FULL_CP, :,
                          pl.ds((t % FULL_CP) * CP_LANES, CP_LANES)]

    def out_dst(t):
        return out_hbm.at[pl.ds((t // FULL_CP) * RPF_PAD
                                + (t % FULL_CP) * RPI, RPI)]

    def compute(in_v, out_v):
        def gath(r):
            return tuple(plsc.load_gather(in_v, [lanes, zeros + (8 * r + k)])
                         for k in range(8))

        def qloop(r, gs):
            gs_new = gath(r)
            for k in range(8):
                out_v[r - 1, pl.ds(16 * k, 16)] = gs[k]
            return gs_new

        gs = lax.fori_loop(1, RPI, qloop, gath(0))
        for k in range(8):
            out_v[RPI - 1, pl.ds(16 * k, 16)] = gs[k]

    pltpu.async_copy(in_src(wid), in_a.at[:, pl.ds(0, CP_LANES)], sia)

    def dbl(j, carry):
        t0 = (2 * j) * NW + wid
        t1 = t0 + NW
        pltpu.async_copy(in_src(t1), in_b.at[:, pl.ds(0, CP_LANES)], sib)
        pltpu.make_async_copy(in_src(t0),
                              in_a.at[:, pl.ds(0, CP_LANES)], sia).wait()
        @pl.when(j > 0)
        def _():
            pltpu.make_async_copy(out_a, out_dst(t0), soa).wait()
        compute(in_a, out_a)
        pltpu.async_copy(out_a, out_dst(t0), soa)
        @pl.when(j < DBL - 1)
        def _():
            pltpu.async_copy(in_src(t0 + 2 * NW),
                             in_a.at[:, pl.ds(0, CP_LANES)], sia)
        pltpu.make_async_copy(in_src(t1),
                              in_b.at[:, pl.ds(0, CP_LANES)], sib).wait()
        @pl.when(j > 0)
        def _():
            pltpu.make_async_copy(out_b, out_dst(t1), sob).wait()
        compute(in_b, out_b)
        pltpu.async_copy(out_b, out_dst(t1), sob)
        return carry

    lax.fori_loop(0, DBL, dbl, 0)
    pltpu.make_async_copy(out_a, out_dst(0), soa).wait()
    pltpu.make_async_copy(out_b, out_dst(0), sob).wait()

    t_last = MAIN_ITEMS * NW + wid
    @pl.when(t_last < N_FULL)
    def _():
        pltpu.sync_copy(in_src(t_last), in_a.at[:, pl.ds(0, CP_LANES)])
        compute(in_a, out_a)
        pltpu.sync_copy(out_a, out_dst(t_last))

    @pl.when(wid < F)
    def _():
        f = wid
        pltpu.sync_copy(
            tail_hbm.at[f],
            out_hbm.at[pl.ds(f * RPF_PAD + FULL_CP * RPI, 24)],
        )


def _transpose_tables(tablesT, tail):
    mesh = plsc.VectorSubcoreMesh(core_axis_name="c", subcore_axis_name="s")
    return pl.kernel(
        _tr_body,
        out_type=jax.ShapeDtypeStruct((F * RPF_PAD, 8 * D), jnp.float32),
        mesh=mesh,
        scratch_types=[
            pltpu.VMEM((D, IN_W), jnp.float32),
            pltpu.VMEM((D, IN_W), jnp.float32),
            pltpu.VMEM((RPI, 8 * D), jnp.float32),
            pltpu.VMEM((RPI, 8 * D), jnp.float32),
            pltpu.SemaphoreType.DMA,
            pltpu.SemaphoreType.DMA,
            pltpu.SemaphoreType.DMA,
            pltpu.SemaphoreType.DMA,
        ],
        compiler_params=pltpu.CompilerParams(
            use_tc_tiling_on_sc=True, needs_layout_passes=False),
    )(tablesT, tail)


def _mlp_body(emb_ref, dense_ref, wbot_ref, bbot_ref, w1a_ref, w1b_ref,
              b1_ref, w2_ref, b2_ref, w3_ref, b3_ref, w4_ref, b4_ref,
              out_ref):
    f32 = jnp.float32
    demb = jnp.dot(dense_ref[...], wbot_ref[...], preferred_element_type=f32)
    demb = demb + bbot_ref[...]
    h = jnp.dot(emb_ref[...], w1a_ref[...], preferred_element_type=f32)
    h = h + jnp.dot(demb, w1b_ref[...], preferred_element_type=f32)
    h = jnp.maximum(h + b1_ref[...], 0.0)
    h = jnp.maximum(jnp.dot(h, w2_ref[...], preferred_element_type=f32) + b2_ref[...], 0.0)
    h = jnp.maximum(jnp.dot(h, w3_ref[...], preferred_element_type=f32) + b3_ref[...], 0.0)
    o = jnp.dot(h, w4_ref[...], preferred_element_type=f32) + b4_ref[...]
    out_ref[...] = jax.nn.sigmoid(o)


_BB = 2048


def _mlp(emb, dense, wbot, bbot, w1a, w1b, b1, w2, b2, w3, b3, w4, b4):
    full = lambda shape: pl.BlockSpec(shape, lambda i: (0, 0))
    return pl.pallas_call(
        _mlp_body,
        grid=(B // _BB,),
        in_specs=[
            pl.BlockSpec((_BB, F * D), lambda i: (i, 0)),
            pl.BlockSpec((_BB, ND), lambda i: (i, 0)),
            full((ND, D)),
            full((1, D)),
            full((F * D, 256)),
            full((D, 256)),
            full((1, 256)),
            full((256, 128)),
            full((1, 128)),
            full((128, 64)),
            full((1, 64)),
            full((64, 1)),
            full((1, 1)),
        ],
        out_specs=pl.BlockSpec((_BB, 1), lambda i: (i, 0)),
        out_shape=jax.ShapeDtypeStruct((B, 1), jnp.float32),
    )(emb, dense, wbot, bbot, w1a, w1b, b1, w2, b2, w3, b3, w4, b4)


def kernel(inputs_sparse, inputs_dense, tables, W_bot, b_bot,
           W1, b1, W2, b2, W3, b3, W4, b4):
    tablesT = jnp.swapaxes(tables, 1, 2)
    tail = jnp.swapaxes(tablesT[:, :, FULL_CP * CP_LANES:], 1, 2)
    tail = jnp.pad(tail.reshape(F, TAIL_LANES // 8, 8 * D),
                   ((0, 0), (0, 4), (0, 0)))
    t3 = _transpose_tables(tablesT, tail).reshape(F, VPAD, D)
    emb = _sc_gather(t3, inputs_sparse.astype(jnp.int32))

    out = _mlp(
        emb, inputs_dense, W_bot, b_bot.reshape(1, D),
        W1[: F * D], W1[F * D:], b1.reshape(1, 256),
        W2, b2.reshape(1, 128), W3, b3.reshape(1, 64),
        W4, b4.reshape(1, 1),
    )
    return out.reshape(-1)
